# baseline (speedup 1.0000x reference)
import functools

import jax
import jax.numpy as jnp
import numpy as np
from jax import lax
from jax.experimental import pallas as pl
from jax.experimental.pallas import tpu as pltpu
from jax.experimental.pallas import tpu_sc as plsc

N_NODES = 10000
N_EDGES = 160000
N_GRAPHS = 256
EMB = 300
HALF = 150
HW = 152
N_LAYERS = 5
OUT_DIM = 2048

NP_NODES = 10240
NB = 20
BLK = 512
PBLK = 512

NSUB = 16
NCORE = 2
CHUNK = 88
LCH = 114
L_PAD = NSUB * LCH * CHUNK - N_EDGES
CCHUNK = 128
CCH = 40
C_PAD = 32 * CCH * CCHUNK - N_EDGES
TREP = 64
DUMP = NP_NODES - 1
SLAB = 640

@functools.lru_cache(maxsize=None)
def _sc_mesh():
    return plsc.VectorSubcoreMesh(core_axis_name="c", subcore_axis_name="s")


def _sc_counts_body(cat_hbm, dstp_hbm, table_hbm, zer_hbm, cpart_hbm,
                    idx_v, didx_v, rows_v, acc_sh, sem):
    c = lax.axis_index("c")
    s = lax.axis_index("s")
    w = c * NSUB + s
    pltpu.sync_copy(zer_hbm, acc_sh.at[pl.ds(s * SLAB, SLAB)])
    pltpu.sync_copy(cat_hbm.at[w], idx_v)
    pltpu.sync_copy(dstp_hbm.at[w], didx_v)
    plsc.subcore_barrier()

    def chunk(j, carry):
        pltpu.async_copy(table_hbm.at[idx_v.at[j]], rows_v, sem).wait()
        pltpu.sync_copy(rows_v, acc_sh.at[didx_v.at[j]], add=True)
        return carry

    lax.fori_loop(0, CCH, chunk, 0)
    plsc.subcore_barrier()
    pltpu.sync_copy(acc_sh.at[pl.ds(s * SLAB, SLAB)],
                    cpart_hbm.at[pl.ds(s * SLAB, SLAB), c])


def _sc_counts_call(catp, dstp, table, zer16):
    return pl.kernel(
        _sc_counts_body,
        out_type=jax.ShapeDtypeStruct((NP_NODES, NCORE, 16), jnp.float32),
        mesh=_sc_mesh(),
        scratch_types=[
            pltpu.VMEM((CCH, CCHUNK), jnp.int32),
            pltpu.VMEM((CCH, CCHUNK), jnp.int32),
            pltpu.VMEM((CCHUNK, 16), jnp.float32),
            pltpu.VMEM_SHARED((NP_NODES, 16), jnp.float32),
            pltpu.SemaphoreType.DMA,
        ],
        compiler_params=pltpu.CompilerParams(use_tc_tiling_on_sc=False),
    )(catp, dstp, table, zer16)


def _sc_layer_body(h2_hbm, src2_hbm, dst3_hbm, zer_hbm, agg_hbm,
                   src_v, dst_v, rows_v, acc_sh, sem):
    c = lax.axis_index("c")
    s = lax.axis_index("s")
    pltpu.sync_copy(zer_hbm, acc_sh.at[pl.ds(s * SLAB, SLAB)])
    pltpu.sync_copy(src2_hbm.at[c, s], src_v)
    pltpu.sync_copy(dst3_hbm.at[s], dst_v)
    plsc.subcore_barrier()

    def chunk(j, carry):
        pltpu.async_copy(h2_hbm.at[src_v.at[j]], rows_v, sem).wait()
        pltpu.sync_copy(rows_v, acc_sh.at[dst_v.at[j]], add=True)
        return carry

    lax.fori_loop(0, LCH, chunk, 0)
    plsc.subcore_barrier()
    pltpu.sync_copy(acc_sh.at[pl.ds(s * SLAB, SLAB)],
                    agg_hbm.at[c, pl.ds(s * SLAB, SLAB)])


def _sc_layer_call(h2, src2, dst3, zer):
    return pl.kernel(
        _sc_layer_body,
        out_type=jax.ShapeDtypeStruct((NCORE, NP_NODES, HW), jnp.float32),
        mesh=_sc_mesh(),
        scratch_types=[
            pltpu.VMEM((LCH, CHUNK), jnp.int32),
            pltpu.VMEM((LCH, CHUNK), jnp.int32),
            pltpu.VMEM((CHUNK, HW), jnp.float32),
            pltpu.VMEM_SHARED((NP_NODES, HW), jnp.float32),
            pltpu.SemaphoreType.DMA,
        ],
        compiler_params=pltpu.CompilerParams(use_tc_tiling_on_sc=False),
    )(h2, src2, dst3, zer)


def _t0_body(atom_ref, chir_ref, aemb_ref, cemb_ref, out_ref):
    a = atom_ref[0, 0, :]
    ch = chir_ref[0, 0, :]
    oa = (a[:, None] == lax.broadcasted_iota(jnp.int32, (BLK, 128), 1)
          ).astype(jnp.float32)
    oc = (ch[:, None] == lax.broadcasted_iota(jnp.int32, (BLK, 8), 1)
          ).astype(jnp.float32)
    h0 = (jnp.dot(oa, aemb_ref[...], preferred_element_type=jnp.float32)
          + jnp.dot(oc, cemb_ref[...], preferred_element_type=jnp.float32))
    z10 = jnp.zeros((BLK, HW - HALF), jnp.float32)
    out_ref[...] = jnp.stack([
        jnp.concatenate([h0[:, :HALF], z10], axis=1),
        jnp.concatenate([h0[:, HALF:], z10], axis=1),
    ])


def _t0_call(atom3, chir3, aembp, cembp):
    return pl.pallas_call(
        _t0_body,
        grid=(NB,),
        in_specs=[
            pl.BlockSpec((1, 1, BLK), lambda b: (b, 0, 0)),
            pl.BlockSpec((1, 1, BLK), lambda b: (b, 0, 0)),
            pl.BlockSpec((128, EMB), lambda b: (0, 0)),
            pl.BlockSpec((8, EMB), lambda b: (0, 0)),
        ],
        out_specs=pl.BlockSpec((NCORE, BLK, HW), lambda b: (0, b, 0)),
        out_shape=jax.ShapeDtypeStruct((NCORE, NP_NODES, HW), jnp.float32),
    )(atom3, chir3, aembp, cembp)


def _mlp_body(agg_ref, ccat_ref, e32_ref, w1_ref, b1_ref, w2_ref, b2_ref,
              out_ref, *, relu_out):
    x = jnp.concatenate([agg_ref[0, :, :HALF], agg_ref[1, :, :HALF]], axis=1)
    x = x + jnp.dot(ccat_ref[...], e32_ref[...],
                    preferred_element_type=jnp.float32)
    y = jnp.dot(x, w1_ref[...], preferred_element_type=jnp.float32)
    y = jnp.maximum(y + b1_ref[...], 0.0)
    h = jnp.dot(y, w2_ref[...], preferred_element_type=jnp.float32)
    h = h + b2_ref[...]
    if relu_out:
        h = jnp.maximum(h, 0.0)
    z10 = jnp.zeros((BLK, HW - HALF), jnp.float32)
    out_ref[...] = jnp.stack([
        jnp.concatenate([h[:, :HALF], z10], axis=1),
        jnp.concatenate([h[:, HALF:], z10], axis=1),
    ])


def _mlp_call(agg, ccat, e32, w1, b1r, w2, b2r, relu_out):
    return pl.pallas_call(
        functools.partial(_mlp_body, relu_out=relu_out),
        grid=(NB,),
        in_specs=[
            pl.BlockSpec((NCORE, BLK, HW), lambda b: (0, b, 0)),
            pl.BlockSpec((BLK, 32), lambda b: (b, 0)),
            pl.BlockSpec((32, EMB), lambda b: (0, 0)),
            pl.BlockSpec((EMB, 2 * EMB), lambda b: (0, 0)),
            pl.BlockSpec((1, 2 * EMB), lambda b: (0, 0)),
            pl.BlockSpec((2 * EMB, EMB), lambda b: (0, 0)),
            pl.BlockSpec((1, EMB), lambda b: (0, 0)),
        ],
        out_specs=pl.BlockSpec((NCORE, BLK, HW), lambda b: (0, b, 0)),
        out_shape=jax.ShapeDtypeStruct((NCORE, NP_NODES, HW), jnp.float32),
    )(agg, ccat, e32, w1, b1r, w2, b2r)


def _pool_body(h_ref, gid_ref, whead_ref, bhead_ref, out_ref, acc_ref):
    b = pl.program_id(0)

    @pl.when(b == 0)
    def _():
        acc_ref[...] = jnp.zeros_like(acc_ref)

    x = jnp.concatenate([h_ref[0, :, :HALF], h_ref[1, :, :HALF]], axis=1)
    g = gid_ref[0, 0, :]
    G = (g[:, None] == lax.broadcasted_iota(jnp.int32, (PBLK, N_GRAPHS), 1)
         ).astype(jnp.float32)
    hx = jnp.concatenate([x, jnp.ones((PBLK, 8), jnp.float32)], axis=1)
    acc_ref[...] += lax.dot_general(G, hx, (((0,), (0,)), ((), ())),
                                    preferred_element_type=jnp.float32)

    @pl.when(b == NB - 1)
    def _():
        sums = acc_ref[:, :EMB]
        cnt = acc_ref[:, EMB:EMB + 1]
        z = sums / jnp.maximum(cnt, 1.0)
        out_ref[...] = (jnp.dot(z, whead_ref[...],
                                preferred_element_type=jnp.float32)
                        + bhead_ref[...])


def _pool_call(h2, gid3, whead, bhead2):
    return pl.pallas_call(
        _pool_body,
        grid=(NB,),
        in_specs=[
            pl.BlockSpec((NCORE, PBLK, HW), lambda b: (0, b, 0)),
            pl.BlockSpec((1, 1, PBLK), lambda b: (b, 0, 0)),
            pl.BlockSpec((EMB, OUT_DIM), lambda b: (0, 0)),
            pl.BlockSpec((1, OUT_DIM), lambda b: (0, 0)),
        ],
        out_specs=pl.BlockSpec((N_GRAPHS, OUT_DIM), lambda b: (0, 0)),
        out_shape=jax.ShapeDtypeStruct((N_GRAPHS, OUT_DIM), jnp.float32),
        scratch_shapes=[pltpu.VMEM((N_GRAPHS, EMB + 8), jnp.float32)],
    )(h2, gid3, whead, bhead2)


_ONEHOT_TABLE = np.zeros((24, 16), np.float32)
for _r in range(18):
    _ONEHOT_TABLE[_r, _r // 3] = 1.0
    _ONEHOT_TABLE[_r, 6 + _r % 3] = 1.0


def kernel(atom_idx, chir_idx, edge_index, bond_idx, dir_idx, graph_ids,
           atom_emb, chir_emb, bond_embs, dir_embs,
           W1s, b1s, W2s, b2s, gammas, betas, bn_means, bn_vars,
           W_head, b_head):
    f32 = jnp.float32
    i32 = jnp.int32

    src = edge_index[0].astype(i32)
    dst = edge_index[1].astype(i32)
    pad_rows = N_NODES + (jnp.arange(L_PAD, dtype=i32) % (NP_NODES - N_NODES))
    src3 = jnp.concatenate(
        [src, jnp.zeros((L_PAD,), i32)]).reshape(NSUB, LCH, CHUNK)
    src2 = jnp.stack([src3, src3 + NP_NODES])
    dst3 = jnp.concatenate([dst, pad_rows]).reshape(NSUB, LCH, CHUNK)
    cat = (bond_idx.astype(i32) * 3 + dir_idx.astype(i32))
    catp = jnp.concatenate([cat, jnp.full((C_PAD,), 18, i32)])
    catp = catp + 24 * (jnp.arange(catp.shape[0], dtype=i32) % TREP)
    catp = catp.reshape(32, CCH, CCHUNK)
    dstp = jnp.concatenate(
        [dst, jnp.full((C_PAD,), DUMP, i32)]).reshape(32, CCH, CCHUNK)
    atom3 = jnp.concatenate(
        [atom_idx.astype(i32), jnp.zeros((NP_NODES - N_NODES,), i32)]
    ).reshape(NB, 1, BLK)
    chir3 = jnp.concatenate(
        [chir_idx.astype(i32), jnp.zeros((NP_NODES - N_NODES,), i32)]
    ).reshape(NB, 1, BLK)
    gid3 = jnp.concatenate(
        [graph_ids.astype(i32),
         jnp.full((NP_NODES - N_NODES,), N_GRAPHS, i32)]).reshape(NB, 1, PBLK)

    aembp = jnp.concatenate([atom_emb.astype(f32),
                             jnp.zeros((8, EMB), f32)], axis=0)
    cembp = jnp.concatenate([chir_emb.astype(f32),
                             jnp.zeros((5, EMB), f32)], axis=0)
    z7 = jnp.zeros((N_LAYERS, 7, EMB), f32)
    e32s = jnp.concatenate(
        [bond_embs, dir_embs, z7, bond_embs, dir_embs, z7], axis=1)
    scale = gammas / jnp.sqrt(bn_vars + 1e-5)
    shift = betas - bn_means * scale
    w2p = W2s * scale[:, None, :]
    b2p = (b2s * scale + shift).reshape(N_LAYERS, 1, EMB)
    b1r = b1s.reshape(N_LAYERS, 1, 2 * EMB)
    bhead2 = b_head.reshape(1, OUT_DIM)

    table = jnp.asarray(np.tile(_ONEHOT_TABLE, (TREP, 1)))
    zer16 = jnp.zeros((SLAB, 16), f32)
    zer = jnp.zeros((SLAB, HW), f32)

    cpart = _sc_counts_call(catp, dstp, table, zer16)
    ccat = cpart.reshape(NP_NODES, 32)
    h2 = _t0_call(atom3, chir3, aembp, cembp)
    for l in range(N_LAYERS):
        agg = _sc_layer_call(h2.reshape(NCORE * NP_NODES, HW), src2, dst3, zer)
        h2 = _mlp_call(agg, ccat, e32s[l], W1s[l], b1r[l], w2p[l], b2p[l],
                       relu_out=(l < N_LAYERS - 1))
    return _pool_call(h2, gid3, W_head, bhead2)


# speedup vs baseline: 1.0406x; 1.0406x over previous
import functools

import jax
import jax.numpy as jnp
import numpy as np
from jax import lax
from jax.experimental import pallas as pl
from jax.experimental.pallas import tpu as pltpu
from jax.experimental.pallas import tpu_sc as plsc

N_NODES = 10000
N_EDGES = 160000
N_GRAPHS = 256
EMB = 300
HALF = 150
HW = 152
N_LAYERS = 5
OUT_DIM = 2048

NP_NODES = 10240
NB = 20
BLK = 512
PBLK = 512

NSUB = 16
NCORE = 2
CHUNK = 80
LCH = 125
CCHUNK = 128
CCH = 40
C_PAD = 32 * CCH * CCHUNK - N_EDGES
TREP = 64
DUMP = NP_NODES - 1
SLAB = 640

@functools.lru_cache(maxsize=None)
def _sc_mesh():
    return plsc.VectorSubcoreMesh(core_axis_name="c", subcore_axis_name="s")


def _sc_counts_body(cat_hbm, dstp_hbm, table_hbm, zer_hbm, cpart_hbm,
                    idx_v, didx_v, rows_v, acc_sh, sem):
    c = lax.axis_index("c")
    s = lax.axis_index("s")
    w = c * NSUB + s
    pltpu.sync_copy(zer_hbm, acc_sh.at[pl.ds(s * SLAB, SLAB)])
    pltpu.sync_copy(cat_hbm.at[w], idx_v)
    pltpu.sync_copy(dstp_hbm.at[w], didx_v)
    plsc.subcore_barrier()

    def chunk(j, carry):
        pltpu.async_copy(table_hbm.at[idx_v.at[j]], rows_v, sem).wait()
        pltpu.sync_copy(rows_v, acc_sh.at[didx_v.at[j]], add=True)
        return carry

    lax.fori_loop(0, CCH, chunk, 0)
    plsc.subcore_barrier()
    pltpu.sync_copy(acc_sh.at[pl.ds(s * SLAB, SLAB)],
                    cpart_hbm.at[pl.ds(s * SLAB, SLAB), c])


def _sc_counts_call(catp, dstp, table, zer16):
    return pl.kernel(
        _sc_counts_body,
        out_type=jax.ShapeDtypeStruct((NP_NODES, NCORE, 16), jnp.float32),
        mesh=_sc_mesh(),
        scratch_types=[
            pltpu.VMEM((CCH, CCHUNK), jnp.int32),
            pltpu.VMEM((CCH, CCHUNK), jnp.int32),
            pltpu.VMEM((CCHUNK, 16), jnp.float32),
            pltpu.VMEM_SHARED((NP_NODES, 16), jnp.float32),
            pltpu.SemaphoreType.DMA,
        ],
        compiler_params=pltpu.CompilerParams(use_tc_tiling_on_sc=False),
    )(catp, dstp, table, zer16)


def _sc_layer_body(h2_hbm, src2_hbm, dst3_hbm, zer_hbm, agg_hbm,
                   src_v, dst_v, rows_v, acc_sh, sem):
    c = lax.axis_index("c")
    s = lax.axis_index("s")
    pltpu.sync_copy(zer_hbm, acc_sh.at[pl.ds(s * SLAB, SLAB)])
    pltpu.sync_copy(src2_hbm.at[c, s], src_v)
    pltpu.sync_copy(dst3_hbm.at[s], dst_v)
    plsc.subcore_barrier()

    def chunk(j, carry):
        pltpu.async_copy(h2_hbm.at[src_v.at[j]], rows_v, sem).wait()
        pltpu.sync_copy(rows_v, acc_sh.at[dst_v.at[j]], add=True)
        return carry

    lax.fori_loop(0, LCH, chunk, 0)
    plsc.subcore_barrier()
    pltpu.sync_copy(acc_sh.at[pl.ds(s * SLAB, SLAB)],
                    agg_hbm.at[c, pl.ds(s * SLAB, SLAB)])


def _sc_layer_call(h2, src2, dst3, zer):
    return pl.kernel(
        _sc_layer_body,
        out_type=jax.ShapeDtypeStruct((NCORE, NP_NODES, HW), jnp.float32),
        mesh=_sc_mesh(),
        scratch_types=[
            pltpu.VMEM((LCH, CHUNK), jnp.int32),
            pltpu.VMEM((LCH, CHUNK), jnp.int32),
            pltpu.VMEM((CHUNK, HW), jnp.float32),
            pltpu.VMEM_SHARED((NP_NODES, HW), jnp.float32),
            pltpu.SemaphoreType.DMA,
        ],
        compiler_params=pltpu.CompilerParams(use_tc_tiling_on_sc=False),
    )(h2, src2, dst3, zer)


def _t0_body(atom_ref, chir_ref, aemb_ref, cemb_ref, out_ref):
    a = atom_ref[0, 0, :]
    ch = chir_ref[0, 0, :]
    oa = (a[:, None] == lax.broadcasted_iota(jnp.int32, (BLK, 128), 1)
          ).astype(jnp.float32)
    oc = (ch[:, None] == lax.broadcasted_iota(jnp.int32, (BLK, 8), 1)
          ).astype(jnp.float32)
    h0 = (jnp.dot(oa, aemb_ref[...], preferred_element_type=jnp.float32)
          + jnp.dot(oc, cemb_ref[...], preferred_element_type=jnp.float32))
    z10 = jnp.zeros((BLK, HW - HALF), jnp.float32)
    out_ref[...] = jnp.stack([
        jnp.concatenate([h0[:, :HALF], z10], axis=1),
        jnp.concatenate([h0[:, HALF:], z10], axis=1),
    ])


def _t0_call(atom3, chir3, aembp, cembp):
    return pl.pallas_call(
        _t0_body,
        grid=(NB,),
        in_specs=[
            pl.BlockSpec((1, 1, BLK), lambda b: (b, 0, 0)),
            pl.BlockSpec((1, 1, BLK), lambda b: (b, 0, 0)),
            pl.BlockSpec((128, EMB), lambda b: (0, 0)),
            pl.BlockSpec((8, EMB), lambda b: (0, 0)),
        ],
        out_specs=pl.BlockSpec((NCORE, BLK, HW), lambda b: (0, b, 0)),
        out_shape=jax.ShapeDtypeStruct((NCORE, NP_NODES, HW), jnp.float32),
    )(atom3, chir3, aembp, cembp)


def _mlp_body(agg_ref, ccat_ref, e32_ref, w1_ref, b1_ref, w2_ref, b2_ref,
              out_ref, *, relu_out):
    x = jnp.concatenate([agg_ref[0, :, :HALF], agg_ref[1, :, :HALF]], axis=1)
    x = x + jnp.dot(ccat_ref[...], e32_ref[...],
                    preferred_element_type=jnp.float32)
    y = jnp.dot(x, w1_ref[...], preferred_element_type=jnp.float32)
    y = jnp.maximum(y + b1_ref[...], 0.0)
    h = jnp.dot(y, w2_ref[...], preferred_element_type=jnp.float32)
    h = h + b2_ref[...]
    if relu_out:
        h = jnp.maximum(h, 0.0)
    z10 = jnp.zeros((BLK, HW - HALF), jnp.float32)
    out_ref[...] = jnp.stack([
        jnp.concatenate([h[:, :HALF], z10], axis=1),
        jnp.concatenate([h[:, HALF:], z10], axis=1),
    ])


def _mlp_call(agg, ccat, e32, w1, b1r, w2, b2r, relu_out):
    return pl.pallas_call(
        functools.partial(_mlp_body, relu_out=relu_out),
        grid=(NB,),
        in_specs=[
            pl.BlockSpec((NCORE, BLK, HW), lambda b: (0, b, 0)),
            pl.BlockSpec((BLK, 32), lambda b: (b, 0)),
            pl.BlockSpec((32, EMB), lambda b: (0, 0)),
            pl.BlockSpec((EMB, 2 * EMB), lambda b: (0, 0)),
            pl.BlockSpec((1, 2 * EMB), lambda b: (0, 0)),
            pl.BlockSpec((2 * EMB, EMB), lambda b: (0, 0)),
            pl.BlockSpec((1, EMB), lambda b: (0, 0)),
        ],
        out_specs=pl.BlockSpec((NCORE, BLK, HW), lambda b: (0, b, 0)),
        out_shape=jax.ShapeDtypeStruct((NCORE, NP_NODES, HW), jnp.float32),
    )(agg, ccat, e32, w1, b1r, w2, b2r)


def _fin_body(agg_ref, ccat_ref, e32_ref, w1_ref, b1_ref, w2_ref, b2_ref,
              gid_ref, whead_ref, bhead_ref, out_ref, acc_ref):
    b = pl.program_id(0)

    @pl.when(b == 0)
    def _():
        acc_ref[...] = jnp.zeros_like(acc_ref)

    x = jnp.concatenate([agg_ref[0, :, :HALF], agg_ref[1, :, :HALF]], axis=1)
    x = x + jnp.dot(ccat_ref[...], e32_ref[...],
                    preferred_element_type=jnp.float32)
    y = jnp.dot(x, w1_ref[...], preferred_element_type=jnp.float32)
    y = jnp.maximum(y + b1_ref[...], 0.0)
    h = jnp.dot(y, w2_ref[...], preferred_element_type=jnp.float32)
    h = h + b2_ref[...]
    g = gid_ref[0, 0, :]
    G = (g[:, None] == lax.broadcasted_iota(jnp.int32, (BLK, N_GRAPHS), 1)
         ).astype(jnp.float32)
    hx = jnp.concatenate([h, jnp.ones((BLK, 8), jnp.float32)], axis=1)
    acc_ref[...] += lax.dot_general(G, hx, (((0,), (0,)), ((), ())),
                                    preferred_element_type=jnp.float32)

    @pl.when(b == NB - 1)
    def _():
        sums = acc_ref[:, :EMB]
        cnt = acc_ref[:, EMB:EMB + 1]
        z = sums / jnp.maximum(cnt, 1.0)
        out_ref[...] = (jnp.dot(z, whead_ref[...],
                                preferred_element_type=jnp.float32)
                        + bhead_ref[...])


def _fin_call(agg, ccat, e32, w1, b1r, w2, b2r, gid3, whead, bhead2):
    return pl.pallas_call(
        _fin_body,
        grid=(NB,),
        in_specs=[
            pl.BlockSpec((NCORE, BLK, HW), lambda b: (0, b, 0)),
            pl.BlockSpec((BLK, 32), lambda b: (b, 0)),
            pl.BlockSpec((32, EMB), lambda b: (0, 0)),
            pl.BlockSpec((EMB, 2 * EMB), lambda b: (0, 0)),
            pl.BlockSpec((1, 2 * EMB), lambda b: (0, 0)),
            pl.BlockSpec((2 * EMB, EMB), lambda b: (0, 0)),
            pl.BlockSpec((1, EMB), lambda b: (0, 0)),
            pl.BlockSpec((1, 1, BLK), lambda b: (b, 0, 0)),
            pl.BlockSpec((EMB, OUT_DIM), lambda b: (0, 0)),
            pl.BlockSpec((1, OUT_DIM), lambda b: (0, 0)),
        ],
        out_specs=pl.BlockSpec((N_GRAPHS, OUT_DIM), lambda b: (0, 0)),
        out_shape=jax.ShapeDtypeStruct((N_GRAPHS, OUT_DIM), jnp.float32),
        scratch_shapes=[pltpu.VMEM((N_GRAPHS, EMB + 8), jnp.float32)],
    )(agg, ccat, e32, w1, b1r, w2, b2r, gid3, whead, bhead2)


_ONEHOT_TABLE = np.zeros((24, 16), np.float32)
for _r in range(18):
    _ONEHOT_TABLE[_r, _r // 3] = 1.0
    _ONEHOT_TABLE[_r, 6 + _r % 3] = 1.0


def kernel(atom_idx, chir_idx, edge_index, bond_idx, dir_idx, graph_ids,
           atom_emb, chir_emb, bond_embs, dir_embs,
           W1s, b1s, W2s, b2s, gammas, betas, bn_means, bn_vars,
           W_head, b_head):
    f32 = jnp.float32
    i32 = jnp.int32

    src = edge_index[0].astype(i32)
    dst = edge_index[1].astype(i32)
    src3 = src.reshape(NSUB, LCH, CHUNK)
    src2 = jnp.stack([src3, src3 + NP_NODES])
    dst3 = dst.reshape(NSUB, LCH, CHUNK)
    cat = (bond_idx.astype(i32) * 3 + dir_idx.astype(i32))
    catp = jnp.concatenate([cat, jnp.full((C_PAD,), 18, i32)])
    catp = catp + 24 * (jnp.arange(catp.shape[0], dtype=i32) % TREP)
    catp = catp.reshape(32, CCH, CCHUNK)
    dstp = jnp.concatenate(
        [dst, jnp.full((C_PAD,), DUMP, i32)]).reshape(32, CCH, CCHUNK)
    atom3 = jnp.concatenate(
        [atom_idx.astype(i32), jnp.zeros((NP_NODES - N_NODES,), i32)]
    ).reshape(NB, 1, BLK)
    chir3 = jnp.concatenate(
        [chir_idx.astype(i32), jnp.zeros((NP_NODES - N_NODES,), i32)]
    ).reshape(NB, 1, BLK)
    gid3 = jnp.concatenate(
        [graph_ids.astype(i32),
         jnp.full((NP_NODES - N_NODES,), N_GRAPHS, i32)]).reshape(NB, 1, PBLK)

    aembp = jnp.concatenate([atom_emb.astype(f32),
                             jnp.zeros((8, EMB), f32)], axis=0)
    cembp = jnp.concatenate([chir_emb.astype(f32),
                             jnp.zeros((5, EMB), f32)], axis=0)
    z7 = jnp.zeros((N_LAYERS, 7, EMB), f32)
    e32s = jnp.concatenate(
        [bond_embs, dir_embs, z7, bond_embs, dir_embs, z7], axis=1)
    scale = gammas / jnp.sqrt(bn_vars + 1e-5)
    shift = betas - bn_means * scale
    w2p = W2s * scale[:, None, :]
    b2p = (b2s * scale + shift).reshape(N_LAYERS, 1, EMB)
    b1r = b1s.reshape(N_LAYERS, 1, 2 * EMB)
    bhead2 = b_head.reshape(1, OUT_DIM)

    table = jnp.asarray(np.tile(_ONEHOT_TABLE, (TREP, 1)))
    zer16 = jnp.zeros((SLAB, 16), f32)
    zer = jnp.zeros((SLAB, HW), f32)

    cpart = _sc_counts_call(catp, dstp, table, zer16)
    ccat = cpart.reshape(NP_NODES, 32)
    h2 = _t0_call(atom3, chir3, aembp, cembp)
    for l in range(N_LAYERS - 1):
        agg = _sc_layer_call(h2.reshape(NCORE * NP_NODES, HW), src2, dst3, zer)
        h2 = _mlp_call(agg, ccat, e32s[l], W1s[l], b1r[l], w2p[l], b2p[l],
                       relu_out=True)
    agg = _sc_layer_call(h2.reshape(NCORE * NP_NODES, HW), src2, dst3, zer)
    l = N_LAYERS - 1
    return _fin_call(agg, ccat, e32s[l], W1s[l], b1r[l], w2p[l], b2p[l],
                     gid3, W_head, bhead2)
